# baseline (device time: 56345 ns/iter reference)
import jax
import jax.numpy as jnp
from jax import lax
from jax.experimental import pallas as pl
from jax.experimental.pallas import tpu as pltpu

N_DEV = 32
N_PAIR = N_DEV // 2
N_RING = 8
N_SEND = 8


def kernel(x, w_mat):
    m_per, k = x.shape
    _, n = w_mat.shape
    n_per = n // N_DEV
    m_out = m_per * N_DEV
    m2 = 2 * m_per

    def body(x_ref, w_hbm, out_ref, xs_ref, wring_ref, sbuf_ref,
             wsems, xsems, send_sems, recv_sems):
        me = lax.axis_index("i")
        parity = lax.rem(me, 2)
        partner = me + 1 - 2 * parity
        mypair = (me - parity) // 2
        pairrow = (me - parity) * m_per

        def wfetch(t):
            dst = lax.rem(me + 2 * t, N_DEV)
            return pltpu.make_async_copy(
                w_hbm.at[:, pl.ds(dst * n_per, n_per)],
                wring_ref.at[t % N_RING],
                wsems.at[t % N_RING],
            )

        xs_ref[pl.ds(parity * m_per, m_per), :] = x_ref[...].astype(jnp.bfloat16)
        for t in range(N_RING):
            wfetch(t).start()

        bsem = pltpu.get_barrier_semaphore()
        pl.semaphore_signal(bsem, inc=1, device_id=(partner,),
                            device_id_type=pl.DeviceIdType.MESH)
        for q in range(1, N_PAIR):
            pl.semaphore_signal(bsem, inc=1,
                                device_id=(lax.rem(me + 2 * q, N_DEV),),
                                device_id_type=pl.DeviceIdType.MESH)
        pl.semaphore_wait(bsem, N_PAIR)

        xrdma = pltpu.make_async_remote_copy(
            src_ref=xs_ref.at[pl.ds(parity * m_per, m_per), :],
            dst_ref=xs_ref.at[pl.ds(parity * m_per, m_per), :],
            send_sem=xsems.at[0],
            recv_sem=xsems.at[1],
            device_id=(partner,),
            device_id_type=pl.DeviceIdType.MESH,
        )
        xrdma.start()
        xrdma.wait_recv()

        for t in range(N_PAIR):
            slot = t % N_RING
            dst = lax.rem(me + 2 * t, N_DEV)
            wfetch(t).wait()
            y = jnp.dot(
                xs_ref[...],
                wring_ref[slot].astype(jnp.bfloat16),
                preferred_element_type=jnp.float32,
            )
            y = jnp.maximum(y, 0.0).astype(jnp.bfloat16)
            if t == 0:
                out_ref[pl.ds(pairrow, m2), :] = y
            else:
                ss = t % N_SEND
                if t >= N_SEND + 1:
                    pltpu.make_async_remote_copy(
                        src_ref=sbuf_ref.at[ss],
                        dst_ref=out_ref.at[pl.ds(pairrow, m2), :],
                        send_sem=send_sems.at[ss],
                        recv_sem=recv_sems.at[mypair],
                        device_id=(dst,),
                        device_id_type=pl.DeviceIdType.MESH,
                    ).wait_send()
                sbuf_ref[ss] = y
                pltpu.make_async_remote_copy(
                    src_ref=sbuf_ref.at[ss],
                    dst_ref=out_ref.at[pl.ds(pairrow, m2), :],
                    send_sem=send_sems.at[ss],
                    recv_sem=recv_sems.at[mypair],
                    device_id=(dst,),
                    device_id_type=pl.DeviceIdType.MESH,
                ).start()
            if t + N_RING < N_PAIR:
                wfetch(t + N_RING).start()

        xrdma.wait_send()
        for t in range(N_PAIR - N_SEND, N_PAIR):
            ss = t % N_SEND
            pltpu.make_async_remote_copy(
                src_ref=sbuf_ref.at[ss],
                dst_ref=out_ref.at[pl.ds(pairrow, m2), :],
                send_sem=send_sems.at[ss],
                recv_sem=recv_sems.at[mypair],
                device_id=(me,),
                device_id_type=pl.DeviceIdType.MESH,
            ).wait_send()
        for q in range(N_PAIR - 1, 0, -1):
            src_pair = lax.rem(mypair + q, N_PAIR)
            pltpu.make_async_remote_copy(
                src_ref=sbuf_ref.at[0],
                dst_ref=out_ref.at[pl.ds(src_pair * m2, m2), :],
                send_sem=send_sems.at[0],
                recv_sem=recv_sems.at[src_pair],
                device_id=(me,),
                device_id_type=pl.DeviceIdType.MESH,
            ).wait_recv()

    return pl.pallas_call(
        body,
        out_shape=jax.ShapeDtypeStruct((m_out, n_per), jnp.bfloat16),
        in_specs=[
            pl.BlockSpec(memory_space=pltpu.MemorySpace.VMEM),
            pl.BlockSpec(memory_space=pl.ANY),
        ],
        out_specs=pl.BlockSpec(memory_space=pltpu.MemorySpace.VMEM),
        scratch_shapes=[
            pltpu.VMEM((m2, k), jnp.bfloat16),
            pltpu.VMEM((N_RING, k, n_per), jnp.float32),
            pltpu.VMEM((N_SEND, m2, n_per), jnp.bfloat16),
            pltpu.SemaphoreType.DMA((N_RING,)),
            pltpu.SemaphoreType.DMA((2,)),
            pltpu.SemaphoreType.DMA((N_SEND,)),
            pltpu.SemaphoreType.DMA((N_PAIR,)),
        ],
        compiler_params=pltpu.CompilerParams(
            collective_id=0,
            vmem_limit_bytes=64 * 1024 * 1024,
        ),
    )(x, w_mat)


# device time: 55030 ns/iter; 1.0239x vs baseline; 1.0239x over previous
import jax
import jax.numpy as jnp
from jax import lax
from jax.experimental import pallas as pl
from jax.experimental.pallas import tpu as pltpu

N_DEV = 32
N_PAIR = N_DEV // 2
N_SEND = 8


def kernel(x, w_mat):
    m_per, k = x.shape
    _, n = w_mat.shape
    n_per = n // N_DEV
    m_out = m_per * N_DEV
    m2 = 2 * m_per

    def body(x_ref, w_ref, out_ref, xs_ref, sbuf_ref,
             xsems, send_sems, recv_sems):
        j = pl.program_id(0)
        me = lax.axis_index("i")
        parity = lax.rem(me, 2)
        partner = me + 1 - 2 * parity
        mypair = (me - parity) // 2
        pairrow = (me - parity) * m_per
        dst = lax.rem(me + 2 * j, N_DEV)

        def xchg():
            return pltpu.make_async_remote_copy(
                src_ref=xs_ref.at[pl.ds(parity * m_per, m_per), :],
                dst_ref=xs_ref.at[pl.ds(parity * m_per, m_per), :],
                send_sem=xsems.at[0],
                recv_sem=xsems.at[1],
                device_id=(partner,),
                device_id_type=pl.DeviceIdType.MESH,
            )

        def result_rdma(slot, dev):
            return pltpu.make_async_remote_copy(
                src_ref=sbuf_ref.at[slot],
                dst_ref=out_ref.at[pl.ds(pairrow, m2), :],
                send_sem=send_sems.at[slot],
                recv_sem=recv_sems.at[mypair],
                device_id=(dev,),
                device_id_type=pl.DeviceIdType.MESH,
            )

        @pl.when(j == 0)
        def _():
            bsem = pltpu.get_barrier_semaphore()
            pl.semaphore_signal(bsem, inc=1, device_id=(partner,),
                                device_id_type=pl.DeviceIdType.MESH)
            for q in range(1, N_PAIR):
                pl.semaphore_signal(bsem, inc=1,
                                    device_id=(lax.rem(me + 2 * q, N_DEV),),
                                    device_id_type=pl.DeviceIdType.MESH)
            pl.semaphore_wait(bsem, N_PAIR)
            xs_ref[pl.ds(parity * m_per, m_per), :] = (
                x_ref[...].astype(jnp.bfloat16))
            ex = xchg()
            ex.start()
            ex.wait_recv()

        y = jnp.dot(
            xs_ref[...],
            w_ref[...].astype(jnp.bfloat16),
            preferred_element_type=jnp.float32,
        )
        y = jnp.maximum(y, 0.0).astype(jnp.bfloat16)

        @pl.when(j == 0)
        def _():
            out_ref[pl.ds(pairrow, m2), :] = y

        @pl.when(j > 0)
        def _():
            slot = lax.rem(j, N_SEND)

            @pl.when(j >= N_SEND + 1)
            def _():
                result_rdma(slot, dst).wait_send()

            sbuf_ref[slot] = y
            result_rdma(slot, dst).start()

        @pl.when(j == N_PAIR - 1)
        def _():
            xchg().wait_send()
            for t in range(N_SEND):
                result_rdma(t, dst).wait_send()
            for q in range(N_PAIR - 1, 0, -1):
                src_pair = lax.rem(mypair + q, N_PAIR)
                pltpu.make_async_remote_copy(
                    src_ref=sbuf_ref.at[0],
                    dst_ref=out_ref.at[pl.ds(src_pair * m2, m2), :],
                    send_sem=send_sems.at[0],
                    recv_sem=recv_sems.at[src_pair],
                    device_id=(me,),
                    device_id_type=pl.DeviceIdType.MESH,
                ).wait_recv()

    grid = (N_PAIR,)
    return pl.pallas_call(
        body,
        grid=grid,
        out_shape=jax.ShapeDtypeStruct((m_out, n_per), jnp.bfloat16),
        in_specs=[
            pl.BlockSpec((m_per, k), lambda j: (0, 0)),
            pl.BlockSpec(
                (k, n_per),
                lambda j: (0, lax.rem(lax.axis_index("i") + 2 * j, N_DEV)),
            ),
        ],
        out_specs=pl.BlockSpec((m_out, n_per), lambda j: (0, 0)),
        scratch_shapes=[
            pltpu.VMEM((m2, k), jnp.bfloat16),
            pltpu.VMEM((N_SEND, m2, n_per), jnp.bfloat16),
            pltpu.SemaphoreType.DMA((2,)),
            pltpu.SemaphoreType.DMA((N_SEND,)),
            pltpu.SemaphoreType.DMA((N_PAIR,)),
        ],
        compiler_params=pltpu.CompilerParams(
            dimension_semantics=("arbitrary",),
            collective_id=0,
        ),
    )(x, w_mat)
